# Initial kernel scaffold; baseline (speedup 1.0000x reference)
import jax
import jax.numpy as jnp
from jax.experimental import pallas as pl


def kernel(x, edge_index, Ws, bs, Wih0, Whh0, bih0, bhh0, Wih1, Whh1, bih1, bhh1, Wl1, bl1, Wr1, Wl2, bl2, Wr2, Wl3, bl3, Wr3, Wh, bh):
    raise NotImplementedError("write your pallas kernel here")


# speedup vs baseline: 3.3183x; 3.3183x over previous
import functools

import jax
import jax.numpy as jnp
from jax import lax
from jax.experimental import pallas as pl
from jax.experimental.pallas import tpu as pltpu
from jax.experimental.pallas import tpu_sc as plsc

F32 = jnp.float32
I32 = jnp.int32

NC, NS = 2, 16
NTILE = NC * NS
NB = 8
RB = 6272
SLAB_ROWS = RB + 16
CHUNK = 128
STAGE = 1280
FLUSH = 1024

_mesh = functools.partial(
    plsc.VectorSubcoreMesh,
    core_axis_name="c", subcore_axis_name="s",
    num_cores=NC, num_subcores=NS,
)


def _extract_i32(vec, lane):
    return jnp.sum(jnp.where(lax.iota(I32, 16) == lane, vec, 0))


def _make_bucket_kernel(E, cap, interpret=False):
    epw = E // NTILE
    nstep = (epw + 15) // 16

    def body(src_hbm, dst_hbm, srcl_hbm, dstl_hbm, cnt_hbm,
             src_in, dst_in, *stages_and_cnt):
        stage_s = stages_and_cnt[:NB]
        stage_d = stages_and_cnt[NB:2 * NB]
        cnt_v = stages_and_cnt[2 * NB]
        c = lax.axis_index("c")
        s = lax.axis_index("s")
        wid = s * NC + c
        base_e = wid * epw
        pltpu.sync_copy(src_hbm.at[pl.ds(pl.multiple_of(base_e, 8), epw)], src_in)
        pltpu.sync_copy(dst_hbm.at[pl.ds(pl.multiple_of(base_e, 8), epw)], dst_in)

        lane = lax.iota(I32, 16)

        def step(i, carry):
            fills = list(carry[:NB])
            ofss = list(carry[NB:])
            b0 = i * 16
            sv = src_in[pl.ds(b0, 16)]
            dv = dst_in[pl.ds(b0, 16)]
            gmask = (b0 + lane) < epw
            for b in range(NB):
                mb = gmask & (dv >= b * RB) & (dv < (b + 1) * RB)
                dvr = dv - b * RB
                prefix = plsc.cumsum(mb.astype(I32))
                pos = fills[b] + prefix - 1
                plsc.store_scatter(stage_s[b], [pos], sv, mask=mb)
                plsc.store_scatter(stage_d[b], [pos], dvr, mask=mb)
                fills[b] = fills[b] + jnp.max(prefix)
                lbase = (b * NTILE + wid) * cap

                def do_flush(fo, b=b, lbase=lbase):
                    f, o = fo
                    pltpu.sync_copy(
                        stage_s[b].at[pl.ds(0, FLUSH)],
                        srcl_hbm.at[pl.ds(pl.multiple_of(lbase + o, 128), FLUSH)])
                    pltpu.sync_copy(
                        stage_d[b].at[pl.ds(0, FLUSH)],
                        dstl_hbm.at[pl.ds(pl.multiple_of(lbase + o, 128), FLUSH)])
                    tail_s = stage_s[b][pl.ds(FLUSH, 16)]
                    tail_d = stage_d[b][pl.ds(FLUSH, 16)]
                    stage_s[b][pl.ds(0, 16)] = tail_s
                    stage_d[b][pl.ds(0, 16)] = tail_d
                    return f - FLUSH, o + FLUSH

                fills[b], ofss[b] = lax.cond(
                    fills[b] >= FLUSH, do_flush, lambda fo: fo,
                    (fills[b], ofss[b]))
            return tuple(fills) + tuple(ofss)

        zero = jnp.zeros((), I32)
        carry = lax.fori_loop(0, nstep, step, (zero,) * (2 * NB))
        fills = carry[:NB]
        ofss = carry[NB:]

        pad_s = jnp.zeros((16,), I32)
        pad_d = jnp.full((16,), RB, I32)
        cvec = jnp.zeros((16,), I32)
        for b in range(NB):
            f = fills[b]
            for j in range(8):
                stage_s[b][pl.ds(f + j * 16, 16)] = pad_s
                stage_d[b][pl.ds(f + j * 16, 16)] = pad_d
            fp = ((f + 127) // 128) * 128
            nblk = fp // 128
            lbase = (b * NTILE + wid) * cap

            def drain(j, o, b=b, lbase=lbase):
                pltpu.sync_copy(
                    stage_s[b].at[pl.ds(j * 128, 128)],
                    srcl_hbm.at[pl.ds(pl.multiple_of(lbase + o + j * 128, 128), 128)])
                pltpu.sync_copy(
                    stage_d[b].at[pl.ds(j * 128, 128)],
                    dstl_hbm.at[pl.ds(pl.multiple_of(lbase + o + j * 128, 128), 128)])
                return o

            lax.fori_loop(0, nblk, drain, ofss[b])
            cvec = jnp.where(lane == b, ofss[b] + fp, cvec)

        cnt_v[...] = cvec
        pltpu.sync_copy(cnt_v, cnt_hbm.at[pl.ds(pl.multiple_of(wid * 16, 16), 16)])

    return pl.kernel(
        body,
        out_type=(
            jax.ShapeDtypeStruct((NB * NTILE * cap,), I32),
            jax.ShapeDtypeStruct((NB * NTILE * cap,), I32),
            jax.ShapeDtypeStruct((NTILE * 16,), I32),
        ),
        mesh=_mesh(),
        compiler_params=pltpu.CompilerParams(needs_layout_passes=False),
        scratch_types=[
            pltpu.VMEM((epw,), I32),
            pltpu.VMEM((epw,), I32),
            *([pltpu.VMEM((STAGE,), I32)] * (2 * NB)),
            pltpu.VMEM((16,), I32),
        ],
        interpret=interpret,
    )


def _make_segsum_kernel(D, mode, cap, cnt_col=None, interpret=False):
    assert D % 16 == 0
    nvec = D // 16
    zrows = SLAB_ROWS // NS
    wrows = RB // NS
    wchunk = 56
    first = mode == "first"

    def body(*refs):
        if first:
            (x_hbm, srcl_hbm, dstl_hbm, cnt_hbm,
             out_hbm, inv_hbm,
             idx_s, idx_d, rows, wbuf, invbuf, cnt_v, slab, sem) = refs
        else:
            (x_hbm, srcl_hbm, dstl_hbm, cnt_hbm, inv_hbm,
             out_hbm,
             idx_s, idx_d, rows, wbuf, invbuf, cnt_v, slab, sem) = refs

        c = lax.axis_index("c")
        s = lax.axis_index("s")
        lane = lax.iota(I32, 16)

        for bloc in range(NB // NC):
            B = c * (NB // NC) + bloc

            def zrow(r, _):
                for j in range(nvec):
                    rows[r, pl.ds(16 * j, 16)] = jnp.zeros((16,), F32)
                return 0
            lax.fori_loop(0, CHUNK, zrow, 0)
            off = 0
            for sz in (CHUNK,) * (zrows // CHUNK) + (zrows % CHUNK,):
                if sz:
                    pltpu.sync_copy(rows.at[pl.ds(0, sz)],
                                    slab.at[pl.ds(s * zrows + off, sz)])
                    off += sz
            plsc.subcore_barrier()

            for t in range(NTILE // NS):
                bt = s * (NTILE // NS) + t
                pltpu.sync_copy(cnt_hbm.at[pl.ds(pl.multiple_of(bt * 16, 16), 16)], cnt_v)
                n = _extract_i32(cnt_v[...], B)
                nblk = n // CHUNK
                lbase = (B * NTILE + bt) * cap

                def kstep(k, _, lbase=lbase):
                    pltpu.sync_copy(
                        srcl_hbm.at[pl.ds(pl.multiple_of(lbase + k * CHUNK, 128), CHUNK)], idx_s)
                    pltpu.sync_copy(
                        dstl_hbm.at[pl.ds(pl.multiple_of(lbase + k * CHUNK, 128), CHUNK)], idx_d)
                    pltpu.async_copy(x_hbm.at[idx_s], rows, sem).wait()
                    pltpu.sync_copy(rows, slab.at[idx_d], add=True)
                    return 0
                lax.fori_loop(0, nblk, kstep, 0)
            plsc.subcore_barrier()

            for cno in range(wrows // wchunk):
                r0 = s * wrows + cno * wchunk
                g0 = B * RB + r0
                pltpu.sync_copy(slab.at[pl.ds(r0, wchunk)], wbuf)
                if not first:
                    pltpu.sync_copy(inv_hbm.at[pl.ds(g0, wchunk)], invbuf)

                def nrow(r, _):
                    if first:
                        cnt = jnp.max(wbuf[r, pl.ds(cnt_col, 16)])
                        cv16 = jnp.full((16,), cnt, F32)
                        iv = 1.0 / jnp.maximum(cv16, 1.0)
                        invbuf[r, pl.ds(0, 16)] = iv
                    else:
                        iv = invbuf[r, pl.ds(0, 16)]
                    for j in range(nvec):
                        wbuf[r, pl.ds(16 * j, 16)] = (
                            wbuf[r, pl.ds(16 * j, 16)] * iv)
                    return 0
                lax.fori_loop(0, wchunk, nrow, 0)
                pltpu.sync_copy(wbuf, out_hbm.at[pl.ds(g0, wchunk)])
                if first:
                    pltpu.sync_copy(invbuf, inv_hbm.at[pl.ds(g0, wchunk)])
            plsc.subcore_barrier()

    out_type = [jax.ShapeDtypeStruct((NB * RB, D), F32)]
    if first:
        out_type.append(jax.ShapeDtypeStruct((NB * RB, 16), F32))

    return pl.kernel(
        body,
        out_type=tuple(out_type),
        mesh=_mesh(),
        compiler_params=pltpu.CompilerParams(needs_layout_passes=False),
        scratch_types=[
            pltpu.VMEM((CHUNK,), I32),
            pltpu.VMEM((CHUNK,), I32),
            pltpu.VMEM((CHUNK, D), F32),
            pltpu.VMEM((wchunk, D), F32),
            pltpu.VMEM((wchunk, 16), F32),
            pltpu.VMEM((16,), I32),
            pltpu.VMEM_SHARED((SLAB_ROWS, D), F32),
            pltpu.SemaphoreType.DMA,
        ],
        interpret=interpret,
    )


TN = 2000


def _mmt(a, b):
    return lax.dot_general(a, b, (((1,), (1,)), ((), ())),
                           preferred_element_type=F32)


def _gru_step(xt_gi, h, whh, bhh):
    gh = _mmt(h, whh) + bhh
    i_r, i_z, i_n = jnp.split(xt_gi, 3, axis=-1)
    h_r, h_z, h_n = jnp.split(gh, 3, axis=-1)
    r = jax.nn.sigmoid(i_r + h_r)
    z = jax.nn.sigmoid(i_z + h_z)
    n = jnp.tanh(i_n + r * h_n)
    return (1.0 - z) * n + z * h


def _enc_body(x_ref, ws_ref, bs_ref, wih0_ref, whh0_ref, bih0_ref, bhh0_ref,
              wih1_ref, whh1_ref, bih1_ref, bhh1_ref, out_ref):
    xb = x_ref[...]
    pre = _mmt(xb[:, :16], ws_ref[...]) + bs_ref[...]
    sfeat = 0.5 * pre * (1.0 + lax.erf(pre * (2.0 ** -0.5)))
    h0 = jnp.zeros((TN, 64), F32)
    h1 = jnp.zeros((TN, 64), F32)
    wih0_row = wih0_ref[...].reshape(1, 192)
    for t in range(6):
        xt = xb[:, 16 + t:17 + t]
        gi0 = xt * wih0_row + bih0_ref[...]
        h0 = _gru_step(gi0, h0, whh0_ref[...], bhh0_ref[...])
        gi1 = _mmt(h0, wih1_ref[...]) + bih1_ref[...]
        h1 = _gru_step(gi1, h1, whh1_ref[...], bhh1_ref[...])
    ones = jnp.ones((TN, 1), F32)
    zeros = jnp.zeros((TN, 31), F32)
    out_ref[...] = jnp.concatenate([h1, sfeat, ones, zeros], axis=-1)


def _conv1_body(agg_ref, x_ref, wl_ref, bl_ref, wr_ref, out_ref):
    h = jnp.maximum(
        _mmt(agg_ref[...][:, :96], wl_ref[...]) + bl_ref[...]
        + _mmt(x_ref[...][:, :96], wr_ref[...]), 0.0)
    out_ref[...] = h


def _conv2_body(agg_ref, x_ref, wl_ref, bl_ref, wr_ref, h2_ref):
    h2_ref[...] = jnp.maximum(
        _mmt(agg_ref[...], wl_ref[...]) + bl_ref[...]
        + _mmt(x_ref[...], wr_ref[...]), 0.0)


def _conv3_body(agg_ref, x_ref, wl3_ref, bl_ref, wr_ref, wh_ref, bh_ref,
                out_ref):
    h3 = jnp.maximum(
        _mmt(agg_ref[...], wl3_ref[...]) + bl_ref[...]
        + _mmt(x_ref[...], wr_ref[...]), 0.0)
    out_ref[...] = jax.nn.sigmoid(_mmt(h3, wh_ref[...]) + bh_ref[0, 0])


def _row_spec(d):
    return pl.BlockSpec((TN, d), lambda i: (i, 0))


def _full_spec(shape):
    nd = len(shape)
    return pl.BlockSpec(shape, lambda i: (0,) * nd)


def _tc_call(body, in_shapes, out_shapes, grid, interpret=False):
    in_specs = [_row_spec(s[1]) if s[0] is None else _full_spec(s)
                for s in in_shapes]
    out_specs = [_row_spec(s[1]) for s in out_shapes]
    out_shape = [jax.ShapeDtypeStruct((grid * TN, s[1]), F32)
                 for s in out_shapes]
    if len(out_shape) == 1:
        out_shape, out_specs = out_shape[0], out_specs[0]
    return pl.pallas_call(
        body, grid=(grid,),
        in_specs=in_specs, out_specs=out_specs, out_shape=out_shape,
        interpret=interpret)


def _forward(x, edge_index, Ws, bs, Wih0, Whh0, bih0, bhh0, Wih1, Whh1,
             bih1, bhh1, Wl1, bl1, Wr1, Wl2, bl2, Wr2, Wl3, bl3, Wr3,
             Wh, bh, interpret=False):
    N = x.shape[0]
    E = edge_index.shape[1]
    grid = N // TN
    cap = ((E // NTILE) + 127) // 128 * 128

    src = edge_index[0].astype(I32)
    dst = edge_index[1].astype(I32)

    enc = _tc_call(
        _enc_body,
        [(None, 34), Ws.shape, (1, 32), Wih0.shape, Whh0.shape, (1, 192),
         (1, 192), Wih1.shape, Whh1.shape, (1, 192), (1, 192)],
        [(None, 128)], grid, interpret)
    h_pad = enc(x, Ws, bs.reshape(1, -1), Wih0, Whh0, bih0.reshape(1, -1),
                bhh0.reshape(1, -1), Wih1, Whh1, bih1.reshape(1, -1),
                bhh1.reshape(1, -1))

    srcl, dstl, cnts = _make_bucket_kernel(E, cap, interpret)(src, dst)

    agg1, inv = _make_segsum_kernel(128, "first", cap, cnt_col=96,
                                    interpret=interpret)(
        h_pad, srcl, dstl, cnts)
    conv1 = _tc_call(_conv1_body,
                     [(None, 128), (None, 128), Wl1.shape, (1, 128),
                      Wr1.shape],
                     [(None, 128)], grid, interpret)
    h1 = conv1(agg1[:N], h_pad, Wl1, bl1.reshape(1, -1), Wr1)

    agg2 = _make_segsum_kernel(128, "inv", cap, interpret=interpret)(
        h1, srcl, dstl, cnts, inv)[0]
    conv2 = _tc_call(_conv2_body,
                     [(None, 128), (None, 128), Wl2.shape, (1, 128),
                      Wr2.shape],
                     [(None, 128)], grid, interpret)
    h2 = conv2(agg2[:N], h1, Wl2, bl2.reshape(1, -1), Wr2)

    agg3 = _make_segsum_kernel(128, "inv", cap, interpret=interpret)(
        h2, srcl, dstl, cnts, inv)[0]
    wh_pad = jnp.zeros((8, Wh.shape[1]), F32).at[0].set(Wh[0])
    conv3 = _tc_call(_conv3_body,
                     [(None, 128), (None, 128), Wl3.shape, (1, 64),
                      Wr3.shape, (8, 64), (1, 1)],
                     [(None, 8)], grid, interpret)
    out = conv3(agg3[:N], h2, Wl3, bl3.reshape(1, -1), Wr3, wh_pad,
                bh.reshape(1, -1))
    return out[:, 0]


def kernel(x, edge_index, Ws, bs, Wih0, Whh0, bih0, bhh0, Wih1, Whh1,
           bih1, bhh1, Wl1, bl1, Wr1, Wl2, bl2, Wr2, Wl3, bl3, Wr3, Wh, bh):
    return _forward(x, edge_index, Ws, bs, Wih0, Whh0, bih0, bhh0, Wih1,
                    Whh1, bih1, bhh1, Wl1, bl1, Wr1, Wl2, bl2, Wr2, Wl3,
                    bl3, Wr3, Wh, bh)


# speedup vs baseline: 4.0841x; 1.2308x over previous
import functools

import jax
import jax.numpy as jnp
from jax import lax
from jax.experimental import pallas as pl
from jax.experimental.pallas import tpu as pltpu
from jax.experimental.pallas import tpu_sc as plsc

F32 = jnp.float32
I32 = jnp.int32

NC, NS = 2, 16
NTILE = NC * NS
NB = 8
RB = 6272
SLAB_ROWS = RB + 16
CHUNK = 128
STAGE = 1280
FLUSH = 1024

_mesh = functools.partial(
    plsc.VectorSubcoreMesh,
    core_axis_name="c", subcore_axis_name="s",
    num_cores=NC, num_subcores=NS,
)


def _extract_i32(vec, lane):
    return jnp.sum(jnp.where(lax.iota(I32, 16) == lane, vec, 0))


def _make_bucket_kernel(E, cap, interpret=False):
    epw = E // NTILE
    nstep = (epw + 15) // 16

    def body(src_hbm, dst_hbm, srcl_hbm, dstl_hbm, cnt_hbm,
             src_in, dst_in, *stages_and_cnt):
        stage_s = stages_and_cnt[:NB]
        stage_d = stages_and_cnt[NB:2 * NB]
        cnt_v = stages_and_cnt[2 * NB]
        c = lax.axis_index("c")
        s = lax.axis_index("s")
        wid = s * NC + c
        base_e = wid * epw
        pltpu.sync_copy(src_hbm.at[pl.ds(pl.multiple_of(base_e, 8), epw)], src_in)
        pltpu.sync_copy(dst_hbm.at[pl.ds(pl.multiple_of(base_e, 8), epw)], dst_in)

        lane = lax.iota(I32, 16)

        def step(i, carry):
            fills = list(carry[:NB])
            ofss = list(carry[NB:])
            b0 = i * 16
            sv = src_in[pl.ds(b0, 16)]
            dv = dst_in[pl.ds(b0, 16)]
            gmask = (b0 + lane) < epw
            for b in range(NB):
                mb = gmask & (dv >= b * RB) & (dv < (b + 1) * RB)
                dvr = dv - b * RB
                prefix = plsc.cumsum(mb.astype(I32))
                pos = fills[b] + prefix - 1
                plsc.store_scatter(stage_s[b], [pos], sv, mask=mb)
                plsc.store_scatter(stage_d[b], [pos], dvr, mask=mb)
                fills[b] = fills[b] + jnp.max(prefix)
                lbase = (b * NTILE + wid) * cap

                def do_flush(fo, b=b, lbase=lbase):
                    f, o = fo
                    pltpu.sync_copy(
                        stage_s[b].at[pl.ds(0, FLUSH)],
                        srcl_hbm.at[pl.ds(pl.multiple_of(lbase + o, 128), FLUSH)])
                    pltpu.sync_copy(
                        stage_d[b].at[pl.ds(0, FLUSH)],
                        dstl_hbm.at[pl.ds(pl.multiple_of(lbase + o, 128), FLUSH)])
                    tail_s = stage_s[b][pl.ds(FLUSH, 16)]
                    tail_d = stage_d[b][pl.ds(FLUSH, 16)]
                    stage_s[b][pl.ds(0, 16)] = tail_s
                    stage_d[b][pl.ds(0, 16)] = tail_d
                    return f - FLUSH, o + FLUSH

                fills[b], ofss[b] = lax.cond(
                    fills[b] >= FLUSH, do_flush, lambda fo: fo,
                    (fills[b], ofss[b]))
            return tuple(fills) + tuple(ofss)

        zero = jnp.zeros((), I32)
        carry = lax.fori_loop(0, nstep, step, (zero,) * (2 * NB))
        fills = carry[:NB]
        ofss = carry[NB:]

        pad_s = jnp.zeros((16,), I32)
        pad_d = jnp.full((16,), RB, I32)
        cvec = jnp.zeros((16,), I32)
        for b in range(NB):
            f = fills[b]
            for j in range(8):
                stage_s[b][pl.ds(f + j * 16, 16)] = pad_s
                stage_d[b][pl.ds(f + j * 16, 16)] = pad_d
            fp = ((f + 127) // 128) * 128
            nblk = fp // 128
            lbase = (b * NTILE + wid) * cap

            def drain(j, o, b=b, lbase=lbase):
                pltpu.sync_copy(
                    stage_s[b].at[pl.ds(j * 128, 128)],
                    srcl_hbm.at[pl.ds(pl.multiple_of(lbase + o + j * 128, 128), 128)])
                pltpu.sync_copy(
                    stage_d[b].at[pl.ds(j * 128, 128)],
                    dstl_hbm.at[pl.ds(pl.multiple_of(lbase + o + j * 128, 128), 128)])
                return o

            lax.fori_loop(0, nblk, drain, ofss[b])
            cvec = jnp.where(lane == b, ofss[b] + fp, cvec)

        cnt_v[...] = cvec
        pltpu.sync_copy(cnt_v, cnt_hbm.at[pl.ds(pl.multiple_of(wid * 16, 16), 16)])

    return pl.kernel(
        body,
        out_type=(
            jax.ShapeDtypeStruct((NB * NTILE * cap,), I32),
            jax.ShapeDtypeStruct((NB * NTILE * cap,), I32),
            jax.ShapeDtypeStruct((NTILE * 16,), I32),
        ),
        mesh=_mesh(),
        compiler_params=pltpu.CompilerParams(needs_layout_passes=False),
        scratch_types=[
            pltpu.VMEM((epw,), I32),
            pltpu.VMEM((epw,), I32),
            *([pltpu.VMEM((STAGE,), I32)] * (2 * NB)),
            pltpu.VMEM((16,), I32),
        ],
        interpret=interpret,
    )


def _make_segsum_kernel(D, mode, cap, cnt_col=None, interpret=False):
    assert D % 16 == 0
    nvec = D // 16
    zrows = SLAB_ROWS // NS
    wrows = RB // NS
    wchunk = 56
    first = mode == "first"

    SUPE = 1024

    def body(*refs):
        if first:
            (x_hbm, srcl_hbm, dstl_hbm, cnt_hbm,
             out_hbm, inv_hbm,
             idx_sb, idx_db, idx_d0, idx_d1, rows0, rows1,
             wbuf, invbuf, cnt_v, slab, sem0, sem1) = refs
        else:
            (x_hbm, srcl_hbm, dstl_hbm, cnt_hbm, inv_hbm,
             out_hbm,
             idx_sb, idx_db, idx_d0, idx_d1, rows0, rows1,
             wbuf, invbuf, cnt_v, slab, sem0, sem1) = refs
        rows = rows0
        rows_bufs = (rows0, rows1)
        idx_dbufs = (idx_d0, idx_d1)
        sems = (sem0, sem1)

        c = lax.axis_index("c")
        s = lax.axis_index("s")
        lane = lax.iota(I32, 16)

        for bloc in range(NB // NC):
            B = c * (NB // NC) + bloc

            def zrow(r, _):
                for j in range(nvec):
                    rows[r, pl.ds(16 * j, 16)] = jnp.zeros((16,), F32)
                return 0
            lax.fori_loop(0, CHUNK, zrow, 0)
            off = 0
            for sz in (CHUNK,) * (zrows // CHUNK) + (zrows % CHUNK,):
                if sz:
                    pltpu.sync_copy(rows.at[pl.ds(0, sz)],
                                    slab.at[pl.ds(s * zrows + off, sz)])
                    off += sz
            plsc.subcore_barrier()

            for t in range(NTILE // NS):
                bt = s * (NTILE // NS) + t
                pltpu.sync_copy(cnt_hbm.at[pl.ds(pl.multiple_of(bt * 16, 16), 16)], cnt_v)
                n = _extract_i32(cnt_v[...], B)
                lbase = (B * NTILE + bt) * cap
                nsuper = n // SUPE
                ntail = (n - nsuper * SUPE) // CHUNK

                def super_body(j, _, lbase=lbase):
                    sbase = pl.multiple_of(lbase + j * SUPE, 128)
                    pltpu.sync_copy(srcl_hbm.at[pl.ds(sbase, SUPE)], idx_sb)
                    pltpu.sync_copy(dstl_hbm.at[pl.ds(sbase, SUPE)], idx_db)
                    for v in range(CHUNK // 16):
                        idx_d0[pl.ds(v * 16, 16)] = idx_db[pl.ds(v * 16, 16)]
                    g = {0: pltpu.async_copy(
                        x_hbm.at[idx_sb.at[pl.ds(0, CHUNK)]], rows0, sem0)}
                    for cc in range(SUPE // CHUNK):
                        cur = cc % 2
                        if cc + 1 < SUPE // CHUNK:
                            nxt = (cc + 1) % 2
                            for v in range(CHUNK // 16):
                                idx_dbufs[nxt][pl.ds(v * 16, 16)] = (
                                    idx_db[pl.ds((cc + 1) * CHUNK + v * 16,
                                                 16)])
                            g[nxt] = pltpu.async_copy(
                                x_hbm.at[idx_sb.at[
                                    pl.ds((cc + 1) * CHUNK, CHUNK)]],
                                rows_bufs[nxt], sems[nxt])
                        g[cur].wait()
                        pltpu.sync_copy(rows_bufs[cur],
                                        slab.at[idx_dbufs[cur]], add=True)
                    return 0
                lax.fori_loop(0, nsuper, super_body, 0)

                def tail_body(k, _, lbase=lbase, nsuper=nsuper):
                    tb = pl.multiple_of(
                        lbase + nsuper * SUPE + k * CHUNK, 128)
                    pltpu.sync_copy(srcl_hbm.at[pl.ds(tb, CHUNK)],
                                    idx_sb.at[pl.ds(0, CHUNK)])
                    pltpu.sync_copy(dstl_hbm.at[pl.ds(tb, CHUNK)], idx_d0)
                    pltpu.async_copy(
                        x_hbm.at[idx_sb.at[pl.ds(0, CHUNK)]], rows0,
                        sem0).wait()
                    pltpu.sync_copy(rows0, slab.at[idx_d0], add=True)
                    return 0
                lax.fori_loop(0, ntail, tail_body, 0)
            plsc.subcore_barrier()

            for cno in range(wrows // wchunk):
                r0 = s * wrows + cno * wchunk
                g0 = B * RB + r0
                pltpu.sync_copy(slab.at[pl.ds(r0, wchunk)], wbuf)
                if not first:
                    pltpu.sync_copy(inv_hbm.at[pl.ds(g0, wchunk)], invbuf)

                def nrow(r, _):
                    if first:
                        cnt = jnp.max(wbuf[r, pl.ds(cnt_col, 16)])
                        cv16 = jnp.full((16,), cnt, F32)
                        iv = 1.0 / jnp.maximum(cv16, 1.0)
                        invbuf[r, pl.ds(0, 16)] = iv
                    else:
                        iv = invbuf[r, pl.ds(0, 16)]
                    for j in range(nvec):
                        wbuf[r, pl.ds(16 * j, 16)] = (
                            wbuf[r, pl.ds(16 * j, 16)] * iv)
                    return 0
                lax.fori_loop(0, wchunk, nrow, 0)
                pltpu.sync_copy(wbuf, out_hbm.at[pl.ds(g0, wchunk)])
                if first:
                    pltpu.sync_copy(invbuf, inv_hbm.at[pl.ds(g0, wchunk)])
            plsc.subcore_barrier()

    out_type = [jax.ShapeDtypeStruct((NB * RB, D), F32)]
    if first:
        out_type.append(jax.ShapeDtypeStruct((NB * RB, 16), F32))

    return pl.kernel(
        body,
        out_type=tuple(out_type),
        mesh=_mesh(),
        compiler_params=pltpu.CompilerParams(needs_layout_passes=False),
        scratch_types=[
            pltpu.VMEM((SUPE,), I32),
            pltpu.VMEM((SUPE,), I32),
            pltpu.VMEM((CHUNK,), I32),
            pltpu.VMEM((CHUNK,), I32),
            pltpu.VMEM((CHUNK, D), F32),
            pltpu.VMEM((CHUNK, D), F32),
            pltpu.VMEM((wchunk, D), F32),
            pltpu.VMEM((wchunk, 16), F32),
            pltpu.VMEM((16,), I32),
            pltpu.VMEM_SHARED((SLAB_ROWS, D), F32),
            pltpu.SemaphoreType.DMA,
            pltpu.SemaphoreType.DMA,
        ],
        interpret=interpret,
    )


TN = 2000


def _mmt(a, b):
    return lax.dot_general(a, b, (((1,), (1,)), ((), ())),
                           preferred_element_type=F32)


def _gru_step(xt_gi, h, whh, bhh):
    gh = _mmt(h, whh) + bhh
    i_r, i_z, i_n = jnp.split(xt_gi, 3, axis=-1)
    h_r, h_z, h_n = jnp.split(gh, 3, axis=-1)
    r = jax.nn.sigmoid(i_r + h_r)
    z = jax.nn.sigmoid(i_z + h_z)
    n = jnp.tanh(i_n + r * h_n)
    return (1.0 - z) * n + z * h


def _enc_body(x_ref, ws_ref, bs_ref, wih0_ref, whh0_ref, bih0_ref, bhh0_ref,
              wih1_ref, whh1_ref, bih1_ref, bhh1_ref, out_ref):
    xb = x_ref[...]
    pre = _mmt(xb[:, :16], ws_ref[...]) + bs_ref[...]
    sfeat = 0.5 * pre * (1.0 + lax.erf(pre * (2.0 ** -0.5)))
    h0 = jnp.zeros((TN, 64), F32)
    h1 = jnp.zeros((TN, 64), F32)
    wih0_row = wih0_ref[...].reshape(1, 192)
    for t in range(6):
        xt = xb[:, 16 + t:17 + t]
        gi0 = xt * wih0_row + bih0_ref[...]
        h0 = _gru_step(gi0, h0, whh0_ref[...], bhh0_ref[...])
        gi1 = _mmt(h0, wih1_ref[...]) + bih1_ref[...]
        h1 = _gru_step(gi1, h1, whh1_ref[...], bhh1_ref[...])
    ones = jnp.ones((TN, 1), F32)
    zeros = jnp.zeros((TN, 31), F32)
    out_ref[...] = jnp.concatenate([h1, sfeat, ones, zeros], axis=-1)


def _conv1_body(agg_ref, x_ref, wl_ref, bl_ref, wr_ref, out_ref):
    h = jnp.maximum(
        _mmt(agg_ref[...][:, :96], wl_ref[...]) + bl_ref[...]
        + _mmt(x_ref[...][:, :96], wr_ref[...]), 0.0)
    out_ref[...] = h


def _conv2_body(agg_ref, x_ref, wl_ref, bl_ref, wr_ref, h2_ref):
    h2_ref[...] = jnp.maximum(
        _mmt(agg_ref[...], wl_ref[...]) + bl_ref[...]
        + _mmt(x_ref[...], wr_ref[...]), 0.0)


def _conv3_body(agg_ref, x_ref, wl3_ref, bl_ref, wr_ref, wh_ref, bh_ref,
                out_ref):
    h3 = jnp.maximum(
        _mmt(agg_ref[...], wl3_ref[...]) + bl_ref[...]
        + _mmt(x_ref[...], wr_ref[...]), 0.0)
    out_ref[...] = jax.nn.sigmoid(_mmt(h3, wh_ref[...]) + bh_ref[0, 0])


def _row_spec(d):
    return pl.BlockSpec((TN, d), lambda i: (i, 0))


def _full_spec(shape):
    nd = len(shape)
    return pl.BlockSpec(shape, lambda i: (0,) * nd)


def _tc_call(body, in_shapes, out_shapes, grid, interpret=False):
    in_specs = [_row_spec(s[1]) if s[0] is None else _full_spec(s)
                for s in in_shapes]
    out_specs = [_row_spec(s[1]) for s in out_shapes]
    out_shape = [jax.ShapeDtypeStruct((grid * TN, s[1]), F32)
                 for s in out_shapes]
    if len(out_shape) == 1:
        out_shape, out_specs = out_shape[0], out_specs[0]
    return pl.pallas_call(
        body, grid=(grid,),
        in_specs=in_specs, out_specs=out_specs, out_shape=out_shape,
        interpret=interpret)


def _forward(x, edge_index, Ws, bs, Wih0, Whh0, bih0, bhh0, Wih1, Whh1,
             bih1, bhh1, Wl1, bl1, Wr1, Wl2, bl2, Wr2, Wl3, bl3, Wr3,
             Wh, bh, interpret=False):
    N = x.shape[0]
    E = edge_index.shape[1]
    grid = N // TN
    cap = ((E // NTILE) + 127) // 128 * 128

    src = edge_index[0].astype(I32)
    dst = edge_index[1].astype(I32)

    enc = _tc_call(
        _enc_body,
        [(None, 34), Ws.shape, (1, 32), Wih0.shape, Whh0.shape, (1, 192),
         (1, 192), Wih1.shape, Whh1.shape, (1, 192), (1, 192)],
        [(None, 128)], grid, interpret)
    h_pad = enc(x, Ws, bs.reshape(1, -1), Wih0, Whh0, bih0.reshape(1, -1),
                bhh0.reshape(1, -1), Wih1, Whh1, bih1.reshape(1, -1),
                bhh1.reshape(1, -1))

    srcl, dstl, cnts = _make_bucket_kernel(E, cap, interpret)(src, dst)

    agg1, inv = _make_segsum_kernel(128, "first", cap, cnt_col=96,
                                    interpret=interpret)(
        h_pad, srcl, dstl, cnts)
    conv1 = _tc_call(_conv1_body,
                     [(None, 128), (None, 128), Wl1.shape, (1, 128),
                      Wr1.shape],
                     [(None, 128)], grid, interpret)
    h1 = conv1(agg1[:N], h_pad, Wl1, bl1.reshape(1, -1), Wr1)

    agg2 = _make_segsum_kernel(128, "inv", cap, interpret=interpret)(
        h1, srcl, dstl, cnts, inv)[0]
    conv2 = _tc_call(_conv2_body,
                     [(None, 128), (None, 128), Wl2.shape, (1, 128),
                      Wr2.shape],
                     [(None, 128)], grid, interpret)
    h2 = conv2(agg2[:N], h1, Wl2, bl2.reshape(1, -1), Wr2)

    agg3 = _make_segsum_kernel(128, "inv", cap, interpret=interpret)(
        h2, srcl, dstl, cnts, inv)[0]
    wh_pad = jnp.zeros((8, Wh.shape[1]), F32).at[0].set(Wh[0])
    conv3 = _tc_call(_conv3_body,
                     [(None, 128), (None, 128), Wl3.shape, (1, 64),
                      Wr3.shape, (8, 64), (1, 1)],
                     [(None, 8)], grid, interpret)
    out = conv3(agg3[:N], h2, Wl3, bl3.reshape(1, -1), Wr3, wh_pad,
                bh.reshape(1, -1))
    return out[:, 0]


def kernel(x, edge_index, Ws, bs, Wih0, Whh0, bih0, bhh0, Wih1, Whh1,
           bih1, bhh1, Wl1, bl1, Wr1, Wl2, bl2, Wr2, Wl3, bl3, Wr3, Wh, bh):
    return _forward(x, edge_index, Ws, bs, Wih0, Whh0, bih0, bhh0, Wih1,
                    Whh1, bih1, bhh1, Wl1, bl1, Wr1, Wl2, bl2, Wr2, Wl3,
                    bl3, Wr3, Wh, bh)
